# baseline (speedup 1.0000x reference)
import jax
import jax.numpy as jnp
from jax.experimental import pallas as pl
from jax.experimental.pallas import tpu as pltpu

_ALPHA = 0.2
_NEG = -9e15


def _gat_body(x_ref, adj_hbm, w_ref, b_ref, a_ref, out_ref, adj_ref, sem):
    c_out = w_ref.shape[0]
    a1 = a_ref[:, :c_out]
    a2 = a_ref[:, c_out:]

    cp = pltpu.make_async_copy(adj_hbm, adj_ref, sem)
    cp.start()

    nf = jax.lax.dot_general(
        x_ref[0], w_ref[...], (((1,), (1,)), ((), ())),
        preferred_element_type=jnp.float32,
    ) + b_ref[...]
    s2 = jax.lax.dot_general(
        a2, nf, (((1,), (1,)), ((), ())),
        preferred_element_type=jnp.float32,
    )
    s1 = jax.lax.dot_general(
        nf, a1, (((1,), (1,)), ((), ())),
        preferred_element_type=jnp.float32,
    )

    cp.wait()
    logits = s1 + s2
    leaky = jnp.maximum(logits, _ALPHA * logits)
    masked = jnp.where(adj_ref[0] != 0, leaky, _NEG)
    m = jnp.max(masked, axis=1, keepdims=True)
    e = jnp.exp(masked - m)
    denom = jnp.sum(e, axis=1, keepdims=True)
    acc = jax.lax.dot_general(
        e, nf, (((1,), (0,)), ((), ())),
        preferred_element_type=jnp.float32,
    )
    out_ref[0] = acc / denom


def kernel(node_feats, adj_matrix, W, b, a):
    if node_feats.ndim == 2:
        node_feats = node_feats[None]
    B, N, C_IN = node_feats.shape
    C_OUT = W.shape[0]
    out = pl.pallas_call(
        _gat_body,
        in_specs=[
            pl.BlockSpec((B, N, C_IN), lambda: (0, 0, 0)),
            pl.BlockSpec(memory_space=pltpu.MemorySpace.HBM),
            pl.BlockSpec((C_OUT, C_IN), lambda: (0, 0)),
            pl.BlockSpec((C_OUT,), lambda: (0,)),
            pl.BlockSpec((1, 2 * C_OUT), lambda: (0, 0)),
        ],
        out_specs=pl.BlockSpec((B, N, C_OUT), lambda: (0, 0, 0)),
        out_shape=jax.ShapeDtypeStruct((B, N, C_OUT), jnp.float32),
        scratch_shapes=[
            pltpu.VMEM((B, N, N), adj_matrix.dtype),
            pltpu.SemaphoreType.DMA,
        ],
    )(node_feats, adj_matrix, W, b, a)
    return out
